# baseline (speedup 1.0000x reference)
import functools

import jax
import jax.numpy as jnp
from jax import lax
from jax.experimental import pallas as pl
from jax.experimental.pallas import tpu as pltpu
from jax.experimental.pallas import tpu_sc as plsc

LANES = 16
NC = 2
NS = 16
NW = NC * NS


def _build(n_tokens, seq_len, hid, K):
    per_w = n_tokens // NW
    n_chunks = per_w // K
    nj = hid // LANES

    mesh = plsc.VectorSubcoreMesh(
        core_axis_name="c", subcore_axis_name="s", num_cores=NC, num_subcores=NS
    )

    @functools.partial(
        pl.kernel,
        out_type=jax.ShapeDtypeStruct((n_tokens, hid), jnp.float32),
        mesh=mesh,
        compiler_params=pltpu.CompilerParams(needs_layout_passes=False),
        scratch_types=[
            pltpu.VMEM((per_w,), jnp.int32),
            pltpu.VMEM((K, hid), jnp.float32),
            pltpu.VMEM((K, hid), jnp.float32),
            pltpu.VMEM((hid,), jnp.float32),
            pltpu.VMEM((hid,), jnp.float32),
            pltpu.SemaphoreType.DMA,
        ],
    )
    def kern(ids_hbm, word_hbm, bias_hbm, lnw_hbm, lnb_hbm, out_hbm,
             idx_v, gbuf, bbuf, lnw_v, lnb_v, sem):
        wid = lax.axis_index("s") * NC + lax.axis_index("c")
        tok0 = wid * per_w
        pos0 = lax.rem(tok0, seq_len)

        pltpu.sync_copy(ids_hbm.at[pl.ds(tok0, per_w)], idx_v)
        pltpu.sync_copy(lnw_hbm, lnw_v)
        pltpu.sync_copy(lnb_hbm, lnb_v)

        def chunk_body(c, _):
            cbase = c * K
            pltpu.async_copy(
                word_hbm.at[idx_v.at[pl.ds(cbase, K)]], gbuf, sem
            ).wait()
            pltpu.sync_copy(bias_hbm.at[pl.ds(pos0 + cbase, K)], bbuf)

            def row_body(r, _):
                def stat_body(j, carry):
                    acc, acc2 = carry
                    sl = pl.ds(j * LANES, LANES)
                    t = gbuf[r, sl] + bbuf[r, sl]
                    gbuf[r, sl] = t
                    return acc + t, acc2 + t * t

                zero = jnp.zeros((LANES,), jnp.float32)
                acc, acc2 = lax.fori_loop(0, nj, stat_body, (zero, zero))
                s1 = jnp.sum(acc)
                s2 = jnp.sum(acc2)
                mean = s1 * (1.0 / hid)
                var = s2 * (1.0 / hid) - mean * mean

                v = jnp.broadcast_to(var + 1e-12, (LANES,)).astype(jnp.float32)
                i = plsc.bitcast(v, jnp.int32)
                i = jnp.int32(0x5F3759DF) - lax.shift_right_logical(i, 1)
                y = plsc.bitcast(i, jnp.float32)
                half = v * 0.5
                for _ in range(4):
                    y = y * (1.5 - half * y * y)
                inv = y
                nm = (-mean) * inv

                def norm_body(j, _):
                    sl = pl.ds(j * LANES, LANES)
                    u = gbuf[r, sl] * inv + nm
                    gbuf[r, sl] = u * lnw_v[sl] + lnb_v[sl]
                    return 0

                lax.fori_loop(0, nj, norm_body, 0)
                return 0

            lax.fori_loop(0, K, row_body, 0)
            pltpu.sync_copy(gbuf, out_hbm.at[pl.ds(tok0 + cbase, K)])
            return 0

        lax.fori_loop(0, n_chunks, chunk_body, 0)

    return kern


@jax.jit
def kernel(input_ids, word_emb, pos_emb, type_emb, ln_weight, ln_bias):
    b, s = input_ids.shape
    hid = word_emb.shape[1]
    ids = input_ids.reshape(-1)
    bias = pos_emb[:s] + type_emb[0][None, :]
    kern = _build(b * s, s, hid, 32)
    out = kern(ids, word_emb, bias, ln_weight, ln_bias)
    return out.reshape(b, s, hid)


# speedup vs baseline: 3.1200x; 3.1200x over previous
import functools

import jax
import jax.numpy as jnp
from jax import lax
from jax.experimental import pallas as pl
from jax.experimental.pallas import tpu as pltpu
from jax.experimental.pallas import tpu_sc as plsc

LANES = 16
NC = 2
NS = 16
NW = NC * NS


def _build(n_tokens, seq_len, hid, K):
    per_w = n_tokens // NW
    n_chunks = per_w // K
    n_pairs = n_chunks // 2
    nj = hid // LANES

    mesh = plsc.VectorSubcoreMesh(
        core_axis_name="c", subcore_axis_name="s", num_cores=NC, num_subcores=NS
    )

    @functools.partial(
        pl.kernel,
        out_type=jax.ShapeDtypeStruct((n_tokens, hid), jnp.float32),
        mesh=mesh,
        compiler_params=pltpu.CompilerParams(needs_layout_passes=False),
        scratch_types=[
            pltpu.VMEM((per_w,), jnp.int32),
            pltpu.VMEM((K, hid), jnp.float32),
            pltpu.VMEM((K, hid), jnp.float32),
            pltpu.VMEM((K, hid), jnp.float32),
            pltpu.VMEM((K, hid), jnp.float32),
            pltpu.VMEM((hid,), jnp.float32),
            pltpu.VMEM((hid,), jnp.float32),
            pltpu.SemaphoreType.DMA,
            pltpu.SemaphoreType.DMA,
            pltpu.SemaphoreType.DMA,
            pltpu.SemaphoreType.DMA,
        ],
    )
    def kern(ids_hbm, word_hbm, bias_hbm, lnw_hbm, lnb_hbm, out_hbm,
             idx_v, g0, g1, b0, b1, lnw_v, lnb_v, is0, is1, os0, os1):
        gbufs = [g0, g1]
        bbufs = [b0, b1]
        isems = [is0, is1]
        osems = [os0, os1]

        wid = lax.axis_index("s") * NC + lax.axis_index("c")
        tok0 = wid * per_w
        pos0 = lax.rem(tok0, seq_len)

        pltpu.sync_copy(ids_hbm.at[pl.ds(tok0, per_w)], idx_v)
        pltpu.sync_copy(lnw_hbm, lnw_v)
        pltpu.sync_copy(lnb_hbm, lnb_v)

        def gather_issue(c, b):
            pltpu.async_copy(
                word_hbm.at[idx_v.at[pl.ds(c * K, K)]], gbufs[b], isems[b]
            )
            pltpu.async_copy(
                bias_hbm.at[pl.ds(pos0 + c * K, K)], bbufs[b], isems[b]
            )

        def gather_wait(b):
            pltpu.make_async_copy(
                word_hbm.at[idx_v.at[pl.ds(0, K)]], gbufs[b], isems[b]
            ).wait()
            pltpu.make_async_copy(
                bias_hbm.at[pl.ds(0, K)], bbufs[b], isems[b]
            ).wait()

        def store_issue(c, b):
            pltpu.async_copy(
                gbufs[b], out_hbm.at[pl.ds(tok0 + c * K, K)], osems[b]
            )

        def store_wait(b):
            pltpu.make_async_copy(
                gbufs[b], out_hbm.at[pl.ds(tok0, K)], osems[b]
            ).wait()

        def ln_row(gb, bb, r):
            acc = [jnp.zeros((LANES,), jnp.float32) for _ in range(4)]
            acc2 = [jnp.zeros((LANES,), jnp.float32) for _ in range(4)]
            for j in range(nj):
                sl = pl.ds(j * LANES, LANES)
                t = gb[r, sl] + bb[r, sl]
                gb[r, sl] = t
                acc[j % 4] = acc[j % 4] + t
                acc2[j % 4] = acc2[j % 4] + t * t
            s1 = jnp.sum((acc[0] + acc[1]) + (acc[2] + acc[3]))
            s2 = jnp.sum((acc2[0] + acc2[1]) + (acc2[2] + acc2[3]))
            mean = s1 * (1.0 / hid)
            var = s2 * (1.0 / hid) - mean * mean

            v = jnp.broadcast_to(var + 1e-12, (LANES,)).astype(jnp.float32)
            i = plsc.bitcast(v, jnp.int32)
            i = jnp.int32(0x5F3759DF) - lax.shift_right_logical(i, 1)
            y = plsc.bitcast(i, jnp.float32)
            half = v * 0.5
            for _ in range(4):
                y = y * (1.5 - half * y * y)
            inv = y
            nm = (-mean) * inv
            for j in range(nj):
                sl = pl.ds(j * LANES, LANES)
                gb[r, sl] = gb[r, sl] * inv + nm
            return

        def compute(b):
            gb = gbufs[b]
            bb = bbufs[b]

            def rows_body(i, _):
                r = i * 2
                ln_row(gb, bb, r)
                ln_row(gb, bb, r + 1)
                return 0

            lax.fori_loop(0, K // 2, rows_body, 0)

            def wb_body(j, _):
                sl = pl.ds(j * LANES, LANES)
                w16 = lnw_v[sl]
                bl16 = lnb_v[sl]
                for r in range(K):
                    gb[r, sl] = gb[r, sl] * w16 + bl16
                return 0

            lax.fori_loop(0, nj, wb_body, 0)

        gather_issue(0, 0)

        def pair_body(p, _):
            c0 = p * 2

            gather_wait(0)

            @pl.when(p >= 1)
            def _():
                store_wait(1)

            gather_issue(c0 + 1, 1)
            compute(0)
            store_issue(c0, 0)

            gather_wait(1)
            store_wait(0)

            @pl.when(p < n_pairs - 1)
            def _():
                gather_issue(c0 + 2, 0)

            compute(1)
            store_issue(c0 + 1, 1)
            return 0

        lax.fori_loop(0, n_pairs, pair_body, 0)
        store_wait(1)

    return kern


@jax.jit
def kernel(input_ids, word_emb, pos_emb, type_emb, ln_weight, ln_bias):
    b, s = input_ids.shape
    hid = word_emb.shape[1]
    ids = input_ids.reshape(-1)
    bias = pos_emb[:s] + type_emb[0][None, :]
    kern = _build(b * s, s, hid, 32)
    out = kern(ids, word_emb, bias, ln_weight, ln_bias)
    return out.reshape(b, s, hid)
